# Initial kernel scaffold; baseline (speedup 1.0000x reference)
import jax
import jax.numpy as jnp
from jax.experimental import pallas as pl


def kernel(means_3d, scales, quats, rgbs, opacities, persistent_mask, indices):
    raise NotImplementedError("write your pallas kernel here")


# speedup vs baseline: 5.3421x; 5.3421x over previous
import functools

import jax
import jax.numpy as jnp
from jax import lax
from jax.experimental import pallas as pl
from jax.experimental.pallas import tpu as pltpu
from jax.experimental.pallas import tpu_sc as plsc


def _sc_hit_kernel(n_rows: int, idx_rows: int):
    ZT = (n_rows // 16 // 8) * 8
    TAIL = n_rows - 16 * ZT
    rows_per_tile = idx_rows // 32

    mesh = plsc.VectorSubcoreMesh(core_axis_name="c", subcore_axis_name="s")

    @functools.partial(
        pl.kernel,
        out_type=(
            jax.ShapeDtypeStruct((n_rows,), jnp.int32),
            jax.ShapeDtypeStruct((n_rows,), jnp.int32),
        ),
        mesh=mesh,
        scratch_types=[
            pltpu.VMEM((ZT,), jnp.int32),
            pltpu.VMEM((rows_per_tile, 128), jnp.int32),
            pltpu.VMEM((128,), jnp.int32),
            pltpu.SemaphoreType.DMA,
        ],
    )
    def hit_kernel(idx_hbm, zeros_hbm, hit0, hit1, zbuf, idxv, ones_v, sem):
        c = lax.axis_index("c")
        s = lax.axis_index("s")

        pltpu.sync_copy(zeros_hbm, zbuf)

        def zero_fill(hit_ref):
            pltpu.sync_copy(zbuf, hit_ref.at[pl.ds(s * ZT, ZT)])
            if TAIL:
                @pl.when(s == 15)
                def _():
                    pltpu.sync_copy(zbuf.at[pl.ds(0, TAIL)],
                                    hit_ref.at[pl.ds(16 * ZT, TAIL)])

        @pl.when(c == 0)
        def _():
            zero_fill(hit0)

        @pl.when(c == 1)
        def _():
            zero_fill(hit1)

        plsc.subcore_barrier()

        for i in range(128 // 16):
            ones_v[pl.ds(i * 16, 16)] = jnp.ones((16,), jnp.int32)

        def scatter(hit_ref, base_row):
            pltpu.sync_copy(
                idx_hbm.at[pl.ds(base_row + s * rows_per_tile, rows_per_tile)],
                idxv)
            handles = [
                pltpu.async_copy(ones_v, hit_ref.at[idxv.at[j]], sem)
                for j in range(rows_per_tile)
            ]
            for h in handles:
                h.wait()

        @pl.when(c == 0)
        def _():
            scatter(hit0, 0)

        @pl.when(c == 1)
        def _():
            scatter(hit1, idx_rows // 2)

    return hit_kernel


def _tc_body(m_i, s_i, q_i, r_i, o_i, pm_i, h0_i, h1_i,
             m_o, s_o, q_o, r_o, o_o, pm_o):
    keep = (h0_i[...] + h1_i[...]) == 0
    kf = keep.astype(jnp.float32)
    m_o[...] = m_i[...] * kf
    s_o[...] = s_i[...] * kf
    q_o[...] = q_i[...] * kf
    r_o[...] = r_i[...] * kf
    o_o[...] = o_i[...] * kf
    pm_o[...] = jnp.logical_and(pm_i[...], keep)


def kernel(means_3d, scales, quats, rgbs, opacities, persistent_mask, indices):
    n = means_3d.shape[0]
    k = indices.shape[0]
    idx2d = indices.astype(jnp.int32).reshape(k // 128, 128)
    zt = (n // 16 // 8) * 8
    zeros_in = jnp.zeros((zt,), jnp.int32)

    hit0, hit1 = _sc_hit_kernel(n, k // 128)(idx2d, zeros_in)

    b = 2000
    grid = (n + b - 1) // b

    def row_spec(w):
        return pl.BlockSpec((b, w), lambda g: (g, 0))

    widths = [means_3d.shape[1], scales.shape[1], quats.shape[1],
              rgbs.shape[1], opacities.shape[1]]
    out_shapes = (
        jax.ShapeDtypeStruct(means_3d.shape, jnp.float32),
        jax.ShapeDtypeStruct(scales.shape, jnp.float32),
        jax.ShapeDtypeStruct(quats.shape, jnp.float32),
        jax.ShapeDtypeStruct(rgbs.shape, jnp.float32),
        jax.ShapeDtypeStruct(opacities.shape, jnp.float32),
        jax.ShapeDtypeStruct((n, 1), jnp.bool_),
    )
    outs = pl.pallas_call(
        _tc_body,
        grid=grid,
        in_specs=[row_spec(w) for w in widths]
        + [row_spec(1), row_spec(1), row_spec(1)],
        out_specs=[row_spec(w) for w in widths] + [row_spec(1)],
        out_shape=out_shapes,
    )(means_3d, scales, quats, rgbs, opacities,
      persistent_mask.reshape(n, 1), hit0.reshape(n, 1), hit1.reshape(n, 1))

    m_o, s_o, q_o, r_o, o_o, pm_o = outs
    return (m_o, s_o, q_o, r_o, o_o, pm_o.reshape(n))


# speedup vs baseline: 121.5498x; 22.7530x over previous
import functools

import jax
import jax.numpy as jnp
from jax import lax
from jax.experimental import pallas as pl
from jax.experimental.pallas import tpu as pltpu
from jax.experimental.pallas import tpu_sc as plsc


def _sc_hit_kernel(n_rows: int, idx_rows: int):
    ZT = (n_rows // 16 // 8) * 8
    TAIL = n_rows - 16 * ZT
    rows_per_tile = idx_rows // 32

    mesh = plsc.VectorSubcoreMesh(core_axis_name="c", subcore_axis_name="s")

    @functools.partial(
        pl.kernel,
        out_type=(
            jax.ShapeDtypeStruct((n_rows,), jnp.int32),
            jax.ShapeDtypeStruct((n_rows,), jnp.int32),
        ),
        mesh=mesh,
        scratch_types=[
            pltpu.VMEM((ZT,), jnp.int32),
            pltpu.VMEM((rows_per_tile, 128), jnp.int32),
            pltpu.VMEM((128,), jnp.int32),
            pltpu.SemaphoreType.DMA,
        ],
    )
    def hit_kernel(idx_hbm, zeros_hbm, hit0, hit1, zbuf, idxv, ones_v, sem):
        c = lax.axis_index("c")
        s = lax.axis_index("s")

        pltpu.sync_copy(zeros_hbm, zbuf)

        def zero_fill(hit_ref):
            pltpu.sync_copy(zbuf, hit_ref.at[pl.ds(s * ZT, ZT)])
            if TAIL:
                @pl.when(s == 15)
                def _():
                    pltpu.sync_copy(zbuf.at[pl.ds(0, TAIL)],
                                    hit_ref.at[pl.ds(16 * ZT, TAIL)])

        @pl.when(c == 0)
        def _():
            zero_fill(hit0)

        @pl.when(c == 1)
        def _():
            zero_fill(hit1)

        plsc.subcore_barrier()

        for i in range(128 // 16):
            ones_v[pl.ds(i * 16, 16)] = jnp.ones((16,), jnp.int32)

        def scatter(hit_ref, base_row):
            pltpu.sync_copy(
                idx_hbm.at[pl.ds(base_row + s * rows_per_tile, rows_per_tile)],
                idxv)
            handles = [
                pltpu.async_copy(ones_v, hit_ref.at[idxv.at[j]], sem)
                for j in range(rows_per_tile)
            ]
            for h in handles:
                h.wait()

        @pl.when(c == 0)
        def _():
            scatter(hit0, 0)

        @pl.when(c == 1)
        def _():
            scatter(hit1, idx_rows // 2)

    return hit_kernel


def _tc_body(m_i, s_i, q_i, r_i, o_i, pm_i, h0_i, h1_i,
             m_o, s_o, q_o, r_o, o_o, pm_o):
    keep = (h0_i[...] + h1_i[...]) == 0
    kf = keep.astype(jnp.float32)
    m_o[...] = m_i[...] * kf
    s_o[...] = s_i[...] * kf
    q_o[...] = q_i[...] * kf
    r_o[...] = r_i[...] * kf
    o_o[...] = o_i[...] * kf
    pm_o[...] = jnp.logical_and(pm_i[...], keep)


def kernel(means_3d, scales, quats, rgbs, opacities, persistent_mask, indices):
    n = means_3d.shape[0]
    k = indices.shape[0]
    idx2d = indices.astype(jnp.int32).reshape(k // 128, 128)
    zt = (n // 16 // 8) * 8
    zeros_in = jnp.zeros((zt,), jnp.int32)

    hit0, hit1 = _sc_hit_kernel(n, k // 128)(idx2d, zeros_in)

    b = 32768
    grid = (n + b - 1) // b

    def col_spec(w):
        return pl.BlockSpec((w, b), lambda g: (0, g))

    widths = [means_3d.shape[1], scales.shape[1], quats.shape[1],
              rgbs.shape[1], opacities.shape[1]]
    out_shapes = (
        jax.ShapeDtypeStruct((widths[0], n), jnp.float32),
        jax.ShapeDtypeStruct((widths[1], n), jnp.float32),
        jax.ShapeDtypeStruct((widths[2], n), jnp.float32),
        jax.ShapeDtypeStruct((widths[3], n), jnp.float32),
        jax.ShapeDtypeStruct((widths[4], n), jnp.float32),
        jax.ShapeDtypeStruct((1, n), jnp.bool_),
    )
    outs = pl.pallas_call(
        _tc_body,
        grid=grid,
        in_specs=[col_spec(w) for w in widths]
        + [col_spec(1), col_spec(1), col_spec(1)],
        out_specs=[col_spec(w) for w in widths] + [col_spec(1)],
        out_shape=out_shapes,
    )(jnp.swapaxes(means_3d, 0, 1), jnp.swapaxes(scales, 0, 1),
      jnp.swapaxes(quats, 0, 1), jnp.swapaxes(rgbs, 0, 1),
      jnp.swapaxes(opacities, 0, 1),
      persistent_mask.reshape(1, n), hit0.reshape(1, n), hit1.reshape(1, n))

    m_o, s_o, q_o, r_o, o_o, pm_o = outs
    return (jnp.swapaxes(m_o, 0, 1), jnp.swapaxes(s_o, 0, 1),
            jnp.swapaxes(q_o, 0, 1), jnp.swapaxes(r_o, 0, 1),
            jnp.swapaxes(o_o, 0, 1), pm_o.reshape(n))


# speedup vs baseline: 124.3022x; 1.0226x over previous
import functools

import jax
import jax.numpy as jnp
from jax import lax
from jax.experimental import pallas as pl
from jax.experimental.pallas import tpu as pltpu
from jax.experimental.pallas import tpu_sc as plsc


def _sc_hit_kernel(n_rows: int, idx_rows: int):
    ZC = 4096
    ZT = (n_rows // 16 // 8) * 8
    TAIL = n_rows - 16 * ZT
    NFULL = ZT // ZC
    ZREM = ZT - NFULL * ZC
    rows_per_tile = idx_rows // 32

    mesh = plsc.VectorSubcoreMesh(core_axis_name="c", subcore_axis_name="s")

    @functools.partial(
        pl.kernel,
        out_type=(
            jax.ShapeDtypeStruct((n_rows,), jnp.int32),
            jax.ShapeDtypeStruct((n_rows,), jnp.int32),
        ),
        mesh=mesh,
        scratch_types=[
            pltpu.VMEM((ZC,), jnp.int32),
            pltpu.VMEM((rows_per_tile, 128), jnp.int32),
            pltpu.VMEM((128,), jnp.int32),
            pltpu.SemaphoreType.DMA,
            pltpu.SemaphoreType.DMA,
        ],
    )
    def hit_kernel(idx_hbm, zeros_hbm, hit0, hit1, zbuf, idxv, ones_v,
                   zsem, sem):
        c = lax.axis_index("c")
        s = lax.axis_index("s")

        pltpu.sync_copy(zeros_hbm, zbuf)

        def zero_fill(hit_ref):
            base = s * ZT
            hs = [
                pltpu.async_copy(
                    zbuf, hit_ref.at[pl.ds(base + j * ZC, ZC)], zsem)
                for j in range(NFULL)
            ]
            if ZREM:
                hs.append(pltpu.async_copy(
                    zbuf.at[pl.ds(0, ZREM)],
                    hit_ref.at[pl.ds(base + NFULL * ZC, ZREM)], zsem))
            if TAIL:
                @pl.when(s == 15)
                def _():
                    pltpu.async_copy(
                        zbuf.at[pl.ds(0, TAIL)],
                        hit_ref.at[pl.ds(16 * ZT, TAIL)], zsem).wait()
            return hs

        @pl.when(c == 0)
        def _():
            for h in zero_fill(hit0):
                h.wait()

        @pl.when(c == 1)
        def _():
            for h in zero_fill(hit1):
                h.wait()

        pltpu.sync_copy(
            idx_hbm.at[pl.ds((c * (idx_rows // 2)) + s * rows_per_tile,
                             rows_per_tile)],
            idxv)
        for i in range(128 // 16):
            ones_v[pl.ds(i * 16, 16)] = jnp.ones((16,), jnp.int32)

        plsc.subcore_barrier()

        def scatter(hit_ref):
            handles = [
                pltpu.async_copy(ones_v, hit_ref.at[idxv.at[j]], sem)
                for j in range(rows_per_tile)
            ]
            for h in handles:
                h.wait()

        @pl.when(c == 0)
        def _():
            scatter(hit0)

        @pl.when(c == 1)
        def _():
            scatter(hit1)

    return hit_kernel


def _tc_body(m_i, s_i, q_i, r_i, o_i, pm_i, h0_i, h1_i,
             m_o, s_o, q_o, r_o, o_o, pm_o):
    keep = (h0_i[...] + h1_i[...]) == 0
    kf = keep.astype(jnp.float32)
    m_o[...] = m_i[...] * kf
    s_o[...] = s_i[...] * kf
    q_o[...] = q_i[...] * kf
    r_o[...] = r_i[...] * kf
    o_o[...] = o_i[...] * kf
    pm_o[...] = jnp.logical_and(pm_i[...], keep)


def kernel(means_3d, scales, quats, rgbs, opacities, persistent_mask, indices):
    n = means_3d.shape[0]
    k = indices.shape[0]
    idx2d = indices.astype(jnp.int32).reshape(k // 128, 128)
    zeros_in = jnp.zeros((4096,), jnp.int32)

    hit0, hit1 = _sc_hit_kernel(n, k // 128)(idx2d, zeros_in)

    b = 32768
    grid = (n + b - 1) // b

    def col_spec(w):
        return pl.BlockSpec((w, b), lambda g: (0, g))

    widths = [means_3d.shape[1], scales.shape[1], quats.shape[1],
              rgbs.shape[1], opacities.shape[1]]
    out_shapes = (
        jax.ShapeDtypeStruct((widths[0], n), jnp.float32),
        jax.ShapeDtypeStruct((widths[1], n), jnp.float32),
        jax.ShapeDtypeStruct((widths[2], n), jnp.float32),
        jax.ShapeDtypeStruct((widths[3], n), jnp.float32),
        jax.ShapeDtypeStruct((widths[4], n), jnp.float32),
        jax.ShapeDtypeStruct((1, n), jnp.bool_),
    )
    outs = pl.pallas_call(
        _tc_body,
        grid=grid,
        in_specs=[col_spec(w) for w in widths]
        + [col_spec(1), col_spec(1), col_spec(1)],
        out_specs=[col_spec(w) for w in widths] + [col_spec(1)],
        out_shape=out_shapes,
    )(jnp.swapaxes(means_3d, 0, 1), jnp.swapaxes(scales, 0, 1),
      jnp.swapaxes(quats, 0, 1), jnp.swapaxes(rgbs, 0, 1),
      jnp.swapaxes(opacities, 0, 1),
      persistent_mask.reshape(1, n), hit0.reshape(1, n), hit1.reshape(1, n))

    m_o, s_o, q_o, r_o, o_o, pm_o = outs
    return (jnp.swapaxes(m_o, 0, 1), jnp.swapaxes(s_o, 0, 1),
            jnp.swapaxes(q_o, 0, 1), jnp.swapaxes(r_o, 0, 1),
            jnp.swapaxes(o_o, 0, 1), pm_o.reshape(n))


# speedup vs baseline: 125.2401x; 1.0075x over previous
import functools

import jax
import jax.numpy as jnp
from jax import lax
from jax.experimental import pallas as pl
from jax.experimental.pallas import tpu as pltpu
from jax.experimental.pallas import tpu_sc as plsc


def _sc_hit_kernel(n_rows: int, idx_rows: int):
    ZC = 4096
    ZT = (n_rows // 16 // 8) * 8
    TAIL = n_rows - 16 * ZT
    NFULL = ZT // ZC
    ZREM = ZT - NFULL * ZC
    rows_per_tile = idx_rows // 32

    mesh = plsc.VectorSubcoreMesh(core_axis_name="c", subcore_axis_name="s")

    @functools.partial(
        pl.kernel,
        out_type=(
            jax.ShapeDtypeStruct((n_rows,), jnp.int32),
            jax.ShapeDtypeStruct((n_rows,), jnp.int32),
        ),
        mesh=mesh,
        scratch_types=[
            pltpu.VMEM((ZC,), jnp.int32),
            pltpu.VMEM((rows_per_tile, 128), jnp.int32),
            pltpu.VMEM((128,), jnp.int32),
            pltpu.SemaphoreType.DMA,
            pltpu.SemaphoreType.DMA,
        ],
    )
    def hit_kernel(idx_hbm, zeros_hbm, hit0, hit1, zbuf, idxv, ones_v,
                   zsem, sem):
        c = lax.axis_index("c")
        s = lax.axis_index("s")

        pltpu.sync_copy(zeros_hbm, zbuf)

        def zero_fill(hit_ref):
            base = s * ZT
            hs = [
                pltpu.async_copy(
                    zbuf, hit_ref.at[pl.ds(base + j * ZC, ZC)], zsem)
                for j in range(NFULL)
            ]
            if ZREM:
                hs.append(pltpu.async_copy(
                    zbuf.at[pl.ds(0, ZREM)],
                    hit_ref.at[pl.ds(base + NFULL * ZC, ZREM)], zsem))
            if TAIL:
                @pl.when(s == 15)
                def _():
                    pltpu.async_copy(
                        zbuf.at[pl.ds(0, TAIL)],
                        hit_ref.at[pl.ds(16 * ZT, TAIL)], zsem).wait()
            return hs

        @pl.when(c == 0)
        def _():
            for h in zero_fill(hit0):
                h.wait()

        @pl.when(c == 1)
        def _():
            for h in zero_fill(hit1):
                h.wait()

        pltpu.sync_copy(
            idx_hbm.at[pl.ds((c * (idx_rows // 2)) + s * rows_per_tile,
                             rows_per_tile)],
            idxv)
        for i in range(128 // 16):
            ones_v[pl.ds(i * 16, 16)] = jnp.ones((16,), jnp.int32)

        plsc.subcore_barrier()

        def scatter(hit_ref):
            handles = [
                pltpu.async_copy(ones_v, hit_ref.at[idxv.at[j]], sem)
                for j in range(rows_per_tile)
            ]
            for h in handles:
                h.wait()

        @pl.when(c == 0)
        def _():
            scatter(hit0)

        @pl.when(c == 1)
        def _():
            scatter(hit1)

    return hit_kernel


def _tc_body(m_i, s_i, q_i, r_i, o_i, pm_i, h0_i, h1_i,
             m_o, s_o, q_o, r_o, o_o, pm_o):
    keep = (h0_i[...] + h1_i[...]) == 0
    kf = keep.astype(jnp.float32)
    m_o[...] = m_i[...] * kf
    s_o[...] = s_i[...] * kf
    q_o[...] = q_i[...] * kf
    r_o[...] = r_i[...] * kf
    o_o[...] = o_i[...] * kf
    pm_o[...] = jnp.logical_and(pm_i[...], keep)


def kernel(means_3d, scales, quats, rgbs, opacities, persistent_mask, indices):
    n = means_3d.shape[0]
    k = indices.shape[0]
    idx2d = indices.astype(jnp.int32).reshape(k // 128, 128)
    zeros_in = jnp.zeros((4096,), jnp.int32)

    hit0, hit1 = _sc_hit_kernel(n, k // 128)(idx2d, zeros_in)

    b = 49152
    grid = (n + b - 1) // b

    def col_spec(w):
        return pl.BlockSpec((w, b), lambda g: (0, g))

    widths = [means_3d.shape[1], scales.shape[1], quats.shape[1],
              rgbs.shape[1], opacities.shape[1]]
    out_shapes = (
        jax.ShapeDtypeStruct((widths[0], n), jnp.float32),
        jax.ShapeDtypeStruct((widths[1], n), jnp.float32),
        jax.ShapeDtypeStruct((widths[2], n), jnp.float32),
        jax.ShapeDtypeStruct((widths[3], n), jnp.float32),
        jax.ShapeDtypeStruct((widths[4], n), jnp.float32),
        jax.ShapeDtypeStruct((1, n), jnp.bool_),
    )
    outs = pl.pallas_call(
        _tc_body,
        grid=grid,
        in_specs=[col_spec(w) for w in widths]
        + [col_spec(1), col_spec(1), col_spec(1)],
        out_specs=[col_spec(w) for w in widths] + [col_spec(1)],
        out_shape=out_shapes,
    )(jnp.swapaxes(means_3d, 0, 1), jnp.swapaxes(scales, 0, 1),
      jnp.swapaxes(quats, 0, 1), jnp.swapaxes(rgbs, 0, 1),
      jnp.swapaxes(opacities, 0, 1),
      persistent_mask.reshape(1, n), hit0.reshape(1, n), hit1.reshape(1, n))

    m_o, s_o, q_o, r_o, o_o, pm_o = outs
    return (jnp.swapaxes(m_o, 0, 1), jnp.swapaxes(s_o, 0, 1),
            jnp.swapaxes(q_o, 0, 1), jnp.swapaxes(r_o, 0, 1),
            jnp.swapaxes(o_o, 0, 1), pm_o.reshape(n))


# speedup vs baseline: 172.9738x; 1.3811x over previous
import functools

import jax
import jax.numpy as jnp
from jax import lax
from jax.experimental import pallas as pl
from jax.experimental.pallas import tpu as pltpu
from jax.experimental.pallas import tpu_sc as plsc


def _sc_hit_kernel(n_rows: int, idx_rows: int):
    H = n_rows // 2
    PAD = 64
    ZC = 4096
    ZT = (H // 16 // 8) * 8
    TAIL = H - 16 * ZT
    NFULL = ZT // ZC
    ZREM = ZT - NFULL * ZC
    rows_per_tile = idx_rows // 16

    mesh = plsc.VectorSubcoreMesh(core_axis_name="c", subcore_axis_name="s")

    @functools.partial(
        pl.kernel,
        out_type=jax.ShapeDtypeStruct((n_rows,), jnp.int32),
        mesh=mesh,
        scratch_types=[
            pltpu.VMEM((ZC,), jnp.int32),
            pltpu.VMEM((rows_per_tile, 128), jnp.int32),
            pltpu.VMEM((128,), jnp.int32),
            pltpu.VMEM((ZT,), jnp.int32),
            pltpu.VMEM_SHARED((H + PAD,), jnp.int32),
            pltpu.SemaphoreType.DMA,
            pltpu.SemaphoreType.DMA,
        ],
    )
    def hit_kernel(idx_hbm, zeros_hbm, hit, zbuf, idxv, ones_v, stage,
                   spbuf, zsem, sem):
        c = lax.axis_index("c")
        s = lax.axis_index("s")
        base = s * ZT

        pltpu.sync_copy(zeros_hbm, zbuf)

        hs = [
            pltpu.async_copy(zbuf, spbuf.at[pl.ds(base + j * ZC, ZC)], zsem)
            for j in range(NFULL)
        ]
        if ZREM:
            hs.append(pltpu.async_copy(
                zbuf.at[pl.ds(0, ZREM)],
                spbuf.at[pl.ds(base + NFULL * ZC, ZREM)], zsem))
        if TAIL:
            @pl.when(s == 15)
            def _():
                pltpu.async_copy(
                    zbuf.at[pl.ds(0, TAIL)],
                    spbuf.at[pl.ds(16 * ZT, TAIL)], zsem).wait()

        pltpu.sync_copy(idx_hbm.at[pl.ds(s * rows_per_tile, rows_per_tile)],
                        idxv)
        lo = c * H
        for r in range(rows_per_tile):
            for i in range(128 // 16):
                v = idxv[r, pl.ds(i * 16, 16)]
                local = v - lo
                ok = (local >= 0) & (local < H)
                idxv[r, pl.ds(i * 16, 16)] = jnp.where(ok, local, H)
        for i in range(128 // 16):
            ones_v[pl.ds(i * 16, 16)] = jnp.ones((16,), jnp.int32)
        for h in hs:
            h.wait()

        plsc.subcore_barrier()

        handles = [
            pltpu.async_copy(ones_v, spbuf.at[idxv.at[j]], sem)
            for j in range(rows_per_tile)
        ]
        for h in handles:
            h.wait()

        plsc.subcore_barrier()

        out_base = c * H + base
        pltpu.sync_copy(spbuf.at[pl.ds(base, ZT)], stage)
        pltpu.sync_copy(stage, hit.at[pl.ds(out_base, ZT)])
        if TAIL:
            @pl.when(s == 15)
            def _():
                pltpu.sync_copy(spbuf.at[pl.ds(16 * ZT, TAIL)],
                                stage.at[pl.ds(0, TAIL)])
                pltpu.sync_copy(stage.at[pl.ds(0, TAIL)],
                                hit.at[pl.ds(c * H + 16 * ZT, TAIL)])

    return hit_kernel


def _tc_body(m_i, s_i, q_i, r_i, o_i, pm_i, h_i,
             m_o, s_o, q_o, r_o, o_o, pm_o):
    keep = h_i[...] == 0
    kf = keep.astype(jnp.float32)
    m_o[...] = m_i[...] * kf
    s_o[...] = s_i[...] * kf
    q_o[...] = q_i[...] * kf
    r_o[...] = r_i[...] * kf
    o_o[...] = o_i[...] * kf
    pm_o[...] = jnp.logical_and(pm_i[...], keep)


def kernel(means_3d, scales, quats, rgbs, opacities, persistent_mask, indices):
    n = means_3d.shape[0]
    k = indices.shape[0]
    idx2d = indices.astype(jnp.int32).reshape(k // 128, 128)
    zeros_in = jnp.zeros((4096,), jnp.int32)

    hit = _sc_hit_kernel(n, k // 128)(idx2d, zeros_in)

    b = 49152
    grid = (n + b - 1) // b

    def col_spec(w):
        return pl.BlockSpec((w, b), lambda g: (0, g))

    widths = [means_3d.shape[1], scales.shape[1], quats.shape[1],
              rgbs.shape[1], opacities.shape[1]]
    out_shapes = (
        jax.ShapeDtypeStruct((widths[0], n), jnp.float32),
        jax.ShapeDtypeStruct((widths[1], n), jnp.float32),
        jax.ShapeDtypeStruct((widths[2], n), jnp.float32),
        jax.ShapeDtypeStruct((widths[3], n), jnp.float32),
        jax.ShapeDtypeStruct((widths[4], n), jnp.float32),
        jax.ShapeDtypeStruct((1, n), jnp.bool_),
    )
    outs = pl.pallas_call(
        _tc_body,
        grid=grid,
        in_specs=[col_spec(w) for w in widths] + [col_spec(1), col_spec(1)],
        out_specs=[col_spec(w) for w in widths] + [col_spec(1)],
        out_shape=out_shapes,
    )(jnp.swapaxes(means_3d, 0, 1), jnp.swapaxes(scales, 0, 1),
      jnp.swapaxes(quats, 0, 1), jnp.swapaxes(rgbs, 0, 1),
      jnp.swapaxes(opacities, 0, 1),
      persistent_mask.reshape(1, n), hit.reshape(1, n))

    m_o, s_o, q_o, r_o, o_o, pm_o = outs
    return (jnp.swapaxes(m_o, 0, 1), jnp.swapaxes(s_o, 0, 1),
            jnp.swapaxes(q_o, 0, 1), jnp.swapaxes(r_o, 0, 1),
            jnp.swapaxes(o_o, 0, 1), pm_o.reshape(n))
